# Initial kernel scaffold; baseline (speedup 1.0000x reference)
import jax
import jax.numpy as jnp
from jax.experimental import pallas as pl


def kernel(old_g_nodes, new_g_nodes, time_map_nodes, weight, bias):
    raise NotImplementedError("write your pallas kernel here")


# speedup vs baseline: 1.0138x; 1.0138x over previous
import jax
import jax.numpy as jnp
from jax.experimental import pallas as pl

_N_FIELDS, _N_NODES, _D_FEAT = 2, 10000, 512


def _copy_body(src_ref, out_ref):
    out_ref[...] = src_ref[...]


def kernel(old_g_nodes, new_g_nodes, time_map_nodes, weight, bias):
    x = new_g_nodes.reshape(_N_FIELDS * _N_NODES, _D_FEAT)
    rows = _N_FIELDS * _N_NODES
    block_rows = 2000
    out = pl.pallas_call(
        _copy_body,
        grid=(rows // block_rows,),
        in_specs=[pl.BlockSpec((block_rows, _D_FEAT), lambda i: (i, 0))],
        out_specs=pl.BlockSpec((block_rows, _D_FEAT), lambda i: (i, 0)),
        out_shape=jax.ShapeDtypeStruct((rows, _D_FEAT), jnp.float32),
    )(x)
    return out.reshape(_N_FIELDS, _N_NODES, _D_FEAT)
